# Initial kernel scaffold; baseline (speedup 1.0000x reference)
import jax
import jax.numpy as jnp
from jax.experimental import pallas as pl


def kernel(x, gate_W, gate_b, W1, b1, W2, b2):
    raise NotImplementedError("write your pallas kernel here")


# speedup vs baseline: 1.0575x; 1.0575x over previous
import jax
import jax.numpy as jnp
from jax.experimental import pallas as pl
from jax.experimental.pallas import tpu as pltpu

LATENT = 1024
FFN = 4096
NE = 8
EPS = 0.01

_GBM = 1024
_FBM = 512


def _gate_body(x_ref, gw_ref, gb_ref, w_ref):
    logits = jnp.dot(x_ref[...].astype(jnp.bfloat16),
                     gw_ref[...].astype(jnp.bfloat16),
                     preferred_element_type=jnp.float32) + gb_ref[...]
    m = jnp.max(logits, axis=-1, keepdims=True)
    p = jnp.exp(logits - m)
    g = p / jnp.sum(p, axis=-1, keepdims=True)
    iota = jax.lax.broadcasted_iota(jnp.int32, g.shape, 1)
    m1 = jnp.max(g, axis=-1, keepdims=True)
    a1 = jnp.min(jnp.where(g == m1, iota, NE), axis=-1, keepdims=True)
    g2 = jnp.where(iota == a1, -jnp.inf, g)
    m2 = jnp.max(g2, axis=-1, keepdims=True)
    a2 = jnp.min(jnp.where(g2 == m2, iota, NE), axis=-1, keepdims=True)
    denom = m1 + m2 + EPS
    w = (jnp.where(iota == a1, m1 / denom, 0.0)
         + jnp.where(iota == a2, m2 / denom, 0.0))
    w_ref[...] = w


def _gate_weights(x_flat, gate_W, gate_b):
    n = x_flat.shape[0]
    return pl.pallas_call(
        _gate_body,
        grid=(n // _GBM,),
        in_specs=[
            pl.BlockSpec((_GBM, LATENT), lambda t: (t, 0)),
            pl.BlockSpec((LATENT, NE), lambda t: (0, 0)),
            pl.BlockSpec((1, NE), lambda t: (0, 0)),
        ],
        out_specs=pl.BlockSpec((_GBM, NE), lambda t: (t, 0)),
        out_shape=jax.ShapeDtypeStruct((n, NE), jnp.float32),
    )(x_flat, gate_W, gate_b.reshape(1, NE))


def _ffn_body(wt_ref, x_ref, w1_ref, b1_ref, w2_ref, b2_ref, out_ref):
    e = pl.program_id(1)
    h = jnp.dot(x_ref[...], w1_ref[0],
                preferred_element_type=jnp.float32) + b1_ref[0]
    h = jnp.maximum(h, 0.0).astype(jnp.bfloat16)
    y = jnp.dot(h, w2_ref[0], preferred_element_type=jnp.float32) + b2_ref[0]
    y = y * wt_ref[0, 0][:, None]

    @pl.when(e == 0)
    def _init():
        out_ref[...] = y

    @pl.when(e > 0)
    def _acc():
        out_ref[...] = out_ref[...] + y


def kernel(x, gate_W, gate_b, W1, b1, W2, b2):
    B, T, D = x.shape
    n = B * T
    x_flat = x.reshape(n, D)
    w_dense = _gate_weights(x_flat, gate_W, gate_b)
    wt = w_dense.T.reshape(NE, 1, n)

    x_bf = x_flat.astype(jnp.bfloat16)
    W1_bf = W1.astype(jnp.bfloat16)
    W2_bf = W2.astype(jnp.bfloat16)

    out = pl.pallas_call(
        _ffn_body,
        grid=(n // _FBM, NE),
        in_specs=[
            pl.BlockSpec((1, 1, _FBM), lambda t, e: (e, 0, t)),
            pl.BlockSpec((_FBM, LATENT), lambda t, e: (t, 0)),
            pl.BlockSpec((1, LATENT, FFN), lambda t, e: (e, 0, 0)),
            pl.BlockSpec((1, 1, FFN), lambda t, e: (e, 0, 0)),
            pl.BlockSpec((1, FFN, LATENT), lambda t, e: (e, 0, 0)),
            pl.BlockSpec((1, 1, LATENT), lambda t, e: (e, 0, 0)),
        ],
        out_specs=pl.BlockSpec((_FBM, LATENT), lambda t, e: (t, 0)),
        out_shape=jax.ShapeDtypeStruct((n, LATENT), jnp.float32),
        compiler_params=pltpu.CompilerParams(
            dimension_semantics=("arbitrary", "arbitrary"),
        ),
    )(wt, x_bf, W1_bf, b1.reshape(NE, 1, FFN), W2_bf, b2.reshape(NE, 1, LATENT))
    return out.reshape(B, T, D)


# speedup vs baseline: 1.8793x; 1.7771x over previous
import functools

import jax
import jax.numpy as jnp
from jax import lax
from jax.experimental import pallas as pl
from jax.experimental.pallas import tpu as pltpu
from jax.experimental.pallas import tpu_sc as plsc

LATENT = 1024
FFN = 4096
NE = 8
EPS = 0.01

CAP = 8192
_GBM = 1024
_FBM = 256
_NBLK = CAP // _FBM
_RTOT = NE * CAP + _FBM

_NSC = 2
_NSUB = 16
_NW = _NSC * _NSUB
_CHUNK = 16


def _gate_body(x_ref, gw_ref, gb_ref, pos1_ref, pos2_ref, w1_ref, w2_ref,
               cnt_ref, base_ref):
    t = pl.program_id(0)

    @pl.when(t == 0)
    def _init():
        base_ref[...] = jnp.zeros_like(base_ref)

    logits = jnp.dot(x_ref[...].astype(jnp.bfloat16),
                     gw_ref[...].astype(jnp.bfloat16),
                     preferred_element_type=jnp.float32) + gb_ref[...]
    m = jnp.max(logits, axis=-1, keepdims=True)
    p = jnp.exp(logits - m)
    g = p / jnp.sum(p, axis=-1, keepdims=True)
    iota = lax.broadcasted_iota(jnp.int32, g.shape, 1)
    m1 = jnp.max(g, axis=-1, keepdims=True)
    a1 = jnp.min(jnp.where(g == m1, iota, NE), axis=-1, keepdims=True)
    g2 = jnp.where(iota == a1, -jnp.inf, g)
    m2 = jnp.max(g2, axis=-1, keepdims=True)
    a2 = jnp.min(jnp.where(g2 == m2, iota, NE), axis=-1, keepdims=True)
    denom = m1 + m2 + EPS
    w1_ref[0, 0, :] = (m1 / denom)[:, 0]
    w2_ref[0, 0, :] = (m2 / denom)[:, 0]

    oh1 = (iota == a1).astype(jnp.float32)
    oh2 = (iota == a2).astype(jnp.float32)
    oh = oh1 + oh2
    ir = lax.broadcasted_iota(jnp.int32, (_GBM, _GBM), 0)
    ic = lax.broadcasted_iota(jnp.int32, (_GBM, _GBM), 1)
    tri = (ir > ic).astype(jnp.bfloat16)
    s = jnp.dot(tri, oh.astype(jnp.bfloat16), preferred_element_type=jnp.float32)
    rank1 = jnp.sum(s * oh1, axis=1).astype(jnp.int32)
    rank2 = jnp.sum(s * oh2, axis=1).astype(jnp.int32)

    base = base_ref[...].astype(jnp.float32)
    base1 = jnp.sum(base * oh1, axis=1).astype(jnp.int32)
    base2 = jnp.sum(base * oh2, axis=1).astype(jnp.int32)
    pos1_ref[0, 0, :] = a1[:, 0] * CAP + base1 + rank1
    pos2_ref[0, 0, :] = a2[:, 0] * CAP + base2 + rank2

    new_base = base_ref[...] + jnp.sum(oh, axis=0, keepdims=True).astype(jnp.int32)
    base_ref[...] = new_base
    cnt_ref[...] = new_base


def _gating(x_flat, gate_W, gate_b):
    n = x_flat.shape[0]
    nb = n // _GBM
    return pl.pallas_call(
        _gate_body,
        grid=(nb,),
        in_specs=[
            pl.BlockSpec((_GBM, LATENT), lambda t: (t, 0)),
            pl.BlockSpec((LATENT, NE), lambda t: (0, 0)),
            pl.BlockSpec((1, NE), lambda t: (0, 0)),
        ],
        out_specs=[
            pl.BlockSpec((1, 1, _GBM), lambda t: (t, 0, 0)),
            pl.BlockSpec((1, 1, _GBM), lambda t: (t, 0, 0)),
            pl.BlockSpec((1, 1, _GBM), lambda t: (t, 0, 0)),
            pl.BlockSpec((1, 1, _GBM), lambda t: (t, 0, 0)),
            pl.BlockSpec((1, NE), lambda t: (0, 0)),
        ],
        out_shape=[
            jax.ShapeDtypeStruct((nb, 1, _GBM), jnp.int32),
            jax.ShapeDtypeStruct((nb, 1, _GBM), jnp.int32),
            jax.ShapeDtypeStruct((nb, 1, _GBM), jnp.float32),
            jax.ShapeDtypeStruct((nb, 1, _GBM), jnp.float32),
            jax.ShapeDtypeStruct((1, NE), jnp.int32),
        ],
        scratch_shapes=[pltpu.VMEM((1, NE), jnp.int32)],
        compiler_params=pltpu.CompilerParams(
            dimension_semantics=("arbitrary",),
        ),
    )(x_flat, gate_W, gate_b.reshape(1, NE))


def _dispatch(x_flat, pos1, pos2, w1, w2):
    n = x_flat.shape[0]
    per_w = n // _NW
    nchunk = per_w // _CHUNK
    mesh = plsc.VectorSubcoreMesh(core_axis_name="c", subcore_axis_name="s")

    @functools.partial(
        pl.kernel, mesh=mesh,
        out_type=[
            jax.ShapeDtypeStruct((_RTOT, LATENT), jnp.float32),
            jax.ShapeDtypeStruct((_RTOT,), jnp.float32),
        ],
        scratch_types=[
            pltpu.VMEM((per_w,), jnp.int32),
            pltpu.VMEM((per_w,), jnp.int32),
            pltpu.VMEM((per_w,), jnp.float32),
            pltpu.VMEM((per_w,), jnp.float32),
            pltpu.VMEM((_CHUNK, LATENT), jnp.float32),
            pltpu.SemaphoreType.DMA,
        ],
    )
    def disp(x_hbm, p1_hbm, p2_hbm, w1_hbm, w2_hbm, xg_hbm, wl_hbm,
             p1_v, p2_v, w1_v, w2_v, rows_v, sem):
        wid = lax.axis_index("s") * _NSC + lax.axis_index("c")
        base = wid * per_w
        pltpu.sync_copy(p1_hbm.at[pl.ds(base, per_w)], p1_v)
        pltpu.sync_copy(p2_hbm.at[pl.ds(base, per_w)], p2_v)
        pltpu.sync_copy(w1_hbm.at[pl.ds(base, per_w)], w1_v)
        pltpu.sync_copy(w2_hbm.at[pl.ds(base, per_w)], w2_v)
        for j in range(nchunk):
            pltpu.sync_copy(x_hbm.at[pl.ds(base + j * _CHUNK, _CHUNK)], rows_v)
            idx1 = p1_v[pl.ds(j * _CHUNK, _CHUNK)]
            idx2 = p2_v[pl.ds(j * _CHUNK, _CHUNK)]
            c1 = pltpu.async_copy(rows_v, xg_hbm.at[idx1], sem)
            c2 = pltpu.async_copy(rows_v, xg_hbm.at[idx2], sem)
            c3 = pltpu.async_copy(w1_v.at[pl.ds(j * _CHUNK, _CHUNK)],
                                  wl_hbm.at[idx1], sem)
            c4 = pltpu.async_copy(w2_v.at[pl.ds(j * _CHUNK, _CHUNK)],
                                  wl_hbm.at[idx2], sem)
            c1.wait()
            c2.wait()
            c3.wait()
            c4.wait()

    return disp(x_flat, pos1, pos2, w1, w2)


def _ffn_body(eb_ref, rb_ref, xg_ref, w1_ref, b1_ref, w2_ref, b2_ref,
              wl_ref, yg_ref):
    h = jnp.dot(xg_ref[...].astype(jnp.bfloat16), w1_ref[0],
                preferred_element_type=jnp.float32) + b1_ref[0]
    h = jnp.maximum(h, 0.0).astype(jnp.bfloat16)
    y = jnp.dot(h, w2_ref[0], preferred_element_type=jnp.float32) + b2_ref[0]
    yg_ref[...] = y * wl_ref[0, 0][:, None]


def _ffn(xg, wl, W1_bf, b1, W2_bf, b2, eb, rb, grid_size):
    grid_spec = pltpu.PrefetchScalarGridSpec(
        num_scalar_prefetch=2,
        grid=(grid_size,),
        in_specs=[
            pl.BlockSpec((_FBM, LATENT), lambda g, eb, rb: (rb[g], 0)),
            pl.BlockSpec((1, LATENT, FFN), lambda g, eb, rb: (eb[g], 0, 0)),
            pl.BlockSpec((1, 1, FFN), lambda g, eb, rb: (eb[g], 0, 0)),
            pl.BlockSpec((1, FFN, LATENT), lambda g, eb, rb: (eb[g], 0, 0)),
            pl.BlockSpec((1, 1, LATENT), lambda g, eb, rb: (eb[g], 0, 0)),
            pl.BlockSpec((1, 1, _FBM), lambda g, eb, rb: (rb[g], 0, 0)),
        ],
        out_specs=pl.BlockSpec((_FBM, LATENT), lambda g, eb, rb: (rb[g], 0)),
    )
    return pl.pallas_call(
        _ffn_body,
        grid_spec=grid_spec,
        out_shape=jax.ShapeDtypeStruct((_RTOT, LATENT), jnp.float32),
        compiler_params=pltpu.CompilerParams(
            dimension_semantics=("arbitrary",),
        ),
    )(eb, rb, xg, W1_bf, b1.reshape(NE, 1, FFN), W2_bf,
      b2.reshape(NE, 1, LATENT), wl.reshape(_RTOT // _FBM, 1, _FBM))


def _combine(yg, pos1, pos2, n):
    per_w = n // _NW
    nchunk = per_w // _CHUNK
    mesh = plsc.VectorSubcoreMesh(core_axis_name="c", subcore_axis_name="s")

    @functools.partial(
        pl.kernel, mesh=mesh,
        out_type=jax.ShapeDtypeStruct((n, LATENT), jnp.float32),
        scratch_types=[
            pltpu.VMEM((per_w,), jnp.int32),
            pltpu.VMEM((per_w,), jnp.int32),
            pltpu.VMEM((_CHUNK, LATENT), jnp.float32),
            pltpu.VMEM((_CHUNK, LATENT), jnp.float32),
            pltpu.SemaphoreType.DMA,
        ],
    )
    def comb(yg_hbm, p1_hbm, p2_hbm, out_hbm, p1_v, p2_v, ra_v, rb_v, sem):
        wid = lax.axis_index("s") * _NSC + lax.axis_index("c")
        base = wid * per_w
        pltpu.sync_copy(p1_hbm.at[pl.ds(base, per_w)], p1_v)
        pltpu.sync_copy(p2_hbm.at[pl.ds(base, per_w)], p2_v)
        for j in range(nchunk):
            idx1 = p1_v[pl.ds(j * _CHUNK, _CHUNK)]
            idx2 = p2_v[pl.ds(j * _CHUNK, _CHUNK)]
            c1 = pltpu.async_copy(yg_hbm.at[idx1], ra_v, sem)
            c2 = pltpu.async_copy(yg_hbm.at[idx2], rb_v, sem)
            c1.wait()
            c2.wait()

            def add_body(i, carry):
                t = i // (LATENT // 16)
                c = i % (LATENT // 16)
                ra_v[t, pl.ds(c * 16, 16)] = (
                    ra_v[t, pl.ds(c * 16, 16)]
                    + rb_v[t, pl.ds(c * 16, 16)])
                return carry

            lax.fori_loop(0, _CHUNK * (LATENT // 16), add_body, 0)
            pltpu.sync_copy(ra_v, out_hbm.at[pl.ds(base + j * _CHUNK, _CHUNK)])

    return comb(yg, pos1, pos2)


def kernel(x, gate_W, gate_b, W1, b1, W2, b2):
    B, T, D = x.shape
    n = B * T
    x_flat = x.reshape(n, D)

    pos1, pos2, w1, w2, counts = _gating(x_flat, gate_W, gate_b)
    pos1 = pos1.reshape(n)
    pos2 = pos2.reshape(n)
    w1 = w1.reshape(n)
    w2 = w2.reshape(n)
    counts = counts.reshape(NE)

    xg, wl = _dispatch(x_flat, pos1, pos2, w1, w2)

    grid_size = 2 * n // _FBM + NE
    nblocks = (counts + (_FBM - 1)) // _FBM
    ends = jnp.cumsum(nblocks)
    starts = ends - nblocks
    total = ends[-1]
    g = jnp.arange(grid_size, dtype=jnp.int32)
    eb = jnp.sum((g[:, None] >= starts[None, :]).astype(jnp.int32), axis=1) - 1
    eb = jnp.clip(eb, 0, NE - 1)
    j = g - starts[eb]
    rb = eb * _NBLK + j
    valid = g < total
    eb_last = eb[jnp.maximum(total - 1, 0)]
    eb = jnp.where(valid, eb, eb_last)
    rb = jnp.where(valid, rb, NE * _NBLK)

    W1_bf = W1.astype(jnp.bfloat16)
    W2_bf = W2.astype(jnp.bfloat16)
    yg = _ffn(xg, wl, W1_bf, b1, W2_bf, b2, eb, rb, grid_size)

    out = _combine(yg, pos1, pos2, n)
    return out.reshape(B, T, D)


# speedup vs baseline: 2.0221x; 1.0760x over previous
import functools

import jax
import jax.numpy as jnp
from jax import lax
from jax.experimental import pallas as pl
from jax.experimental.pallas import tpu as pltpu
from jax.experimental.pallas import tpu_sc as plsc

LATENT = 1024
FFN = 4096
NE = 8
EPS = 0.01

CAP = 8192
_GBM = 1024
_FBM = 256
_NBLK = CAP // _FBM
_RTOT = NE * CAP + _FBM

_NSC = 2
_NSUB = 16
_NW = _NSC * _NSUB
_CHUNK = 16


def _gate_body(x_ref, gw_ref, gb_ref, pos1_ref, pos2_ref, w1_ref, w2_ref,
               cnt_ref, base_ref):
    t = pl.program_id(0)

    @pl.when(t == 0)
    def _init():
        base_ref[...] = jnp.zeros_like(base_ref)

    logits = jnp.dot(x_ref[...].astype(jnp.bfloat16),
                     gw_ref[...].astype(jnp.bfloat16),
                     preferred_element_type=jnp.float32) + gb_ref[...]
    m = jnp.max(logits, axis=-1, keepdims=True)
    p = jnp.exp(logits - m)
    g = p / jnp.sum(p, axis=-1, keepdims=True)
    iota = lax.broadcasted_iota(jnp.int32, g.shape, 1)
    m1 = jnp.max(g, axis=-1, keepdims=True)
    a1 = jnp.min(jnp.where(g == m1, iota, NE), axis=-1, keepdims=True)
    g2 = jnp.where(iota == a1, -jnp.inf, g)
    m2 = jnp.max(g2, axis=-1, keepdims=True)
    a2 = jnp.min(jnp.where(g2 == m2, iota, NE), axis=-1, keepdims=True)
    denom = m1 + m2 + EPS
    w1_ref[0, 0, :] = (m1 / denom)[:, 0]
    w2_ref[0, 0, :] = (m2 / denom)[:, 0]

    oh1 = (iota == a1).astype(jnp.float32)
    oh2 = (iota == a2).astype(jnp.float32)
    oh = oh1 + oh2
    ir = lax.broadcasted_iota(jnp.int32, (_GBM, _GBM), 0)
    ic = lax.broadcasted_iota(jnp.int32, (_GBM, _GBM), 1)
    tri = (ir > ic).astype(jnp.bfloat16)
    s = jnp.dot(tri, oh.astype(jnp.bfloat16), preferred_element_type=jnp.float32)
    rank1 = jnp.sum(s * oh1, axis=1).astype(jnp.int32)
    rank2 = jnp.sum(s * oh2, axis=1).astype(jnp.int32)

    base = base_ref[...].astype(jnp.float32)
    base1 = jnp.sum(base * oh1, axis=1).astype(jnp.int32)
    base2 = jnp.sum(base * oh2, axis=1).astype(jnp.int32)
    pos1_ref[0, 0, :] = a1[:, 0] * CAP + base1 + rank1
    pos2_ref[0, 0, :] = a2[:, 0] * CAP + base2 + rank2

    new_base = base_ref[...] + jnp.sum(oh, axis=0, keepdims=True).astype(jnp.int32)
    base_ref[...] = new_base
    cnt_ref[...] = new_base


def _gating(x_flat, gate_W, gate_b):
    n = x_flat.shape[0]
    nb = n // _GBM
    return pl.pallas_call(
        _gate_body,
        grid=(nb,),
        in_specs=[
            pl.BlockSpec((_GBM, LATENT), lambda t: (t, 0)),
            pl.BlockSpec((LATENT, NE), lambda t: (0, 0)),
            pl.BlockSpec((1, NE), lambda t: (0, 0)),
        ],
        out_specs=[
            pl.BlockSpec((1, 1, _GBM), lambda t: (t, 0, 0)),
            pl.BlockSpec((1, 1, _GBM), lambda t: (t, 0, 0)),
            pl.BlockSpec((1, 1, _GBM), lambda t: (t, 0, 0)),
            pl.BlockSpec((1, 1, _GBM), lambda t: (t, 0, 0)),
            pl.BlockSpec((1, NE), lambda t: (0, 0)),
        ],
        out_shape=[
            jax.ShapeDtypeStruct((nb, 1, _GBM), jnp.int32),
            jax.ShapeDtypeStruct((nb, 1, _GBM), jnp.int32),
            jax.ShapeDtypeStruct((nb, 1, _GBM), jnp.float32),
            jax.ShapeDtypeStruct((nb, 1, _GBM), jnp.float32),
            jax.ShapeDtypeStruct((1, NE), jnp.int32),
        ],
        scratch_shapes=[pltpu.VMEM((1, NE), jnp.int32)],
        compiler_params=pltpu.CompilerParams(
            dimension_semantics=("arbitrary",),
        ),
    )(x_flat, gate_W, gate_b.reshape(1, NE))


def _dispatch(x_flat, pos1, pos2, w1, w2):
    n = x_flat.shape[0]
    per_w = n // _NW
    nchunk = per_w // _CHUNK
    mesh = plsc.VectorSubcoreMesh(core_axis_name="c", subcore_axis_name="s")

    nbuf = 4

    @functools.partial(
        pl.kernel, mesh=mesh,
        out_type=[
            jax.ShapeDtypeStruct((_RTOT, LATENT), jnp.float32),
            jax.ShapeDtypeStruct((_RTOT,), jnp.float32),
        ],
        scratch_types=[
            pltpu.VMEM((per_w,), jnp.int32),
            pltpu.VMEM((per_w,), jnp.int32),
            pltpu.VMEM((per_w,), jnp.float32),
            pltpu.VMEM((per_w,), jnp.float32),
        ] + [pltpu.VMEM((_CHUNK, LATENT), jnp.float32) for _ in range(nbuf)]
          + [pltpu.SemaphoreType.DMA, pltpu.SemaphoreType.DMA,
             pltpu.SemaphoreType.DMA],
    )
    def disp(x_hbm, p1_hbm, p2_hbm, w1_hbm, w2_hbm, xg_hbm, wl_hbm,
             p1_v, p2_v, w1_v, w2_v, *bufs_and_sems):
        rows = bufs_and_sems[:nbuf]
        rsem, ssem, wsem = bufs_and_sems[nbuf:]
        wid = lax.axis_index("s") * _NSC + lax.axis_index("c")
        base = wid * per_w
        pltpu.sync_copy(p1_hbm.at[pl.ds(base, per_w)], p1_v)
        pltpu.sync_copy(p2_hbm.at[pl.ds(base, per_w)], p2_v)
        pltpu.sync_copy(w1_hbm.at[pl.ds(base, per_w)], w1_v)
        pltpu.sync_copy(w2_hbm.at[pl.ds(base, per_w)], w2_v)
        cw1 = pltpu.async_copy(w1_v, wl_hbm.at[p1_v], wsem)
        cw2 = pltpu.async_copy(w2_v, wl_hbm.at[p2_v], wsem)

        reads = [None] * nchunk
        scats = [None] * nchunk
        for j in range(min(nbuf, nchunk)):
            reads[j] = pltpu.async_copy(
                x_hbm.at[pl.ds(base + j * _CHUNK, _CHUNK)], rows[j % nbuf],
                rsem)
        for j in range(nchunk):
            reads[j].wait()
            idx1 = p1_v[pl.ds(j * _CHUNK, _CHUNK)]
            idx2 = p2_v[pl.ds(j * _CHUNK, _CHUNK)]
            b = rows[j % nbuf]
            scats[j] = (pltpu.async_copy(b, xg_hbm.at[idx1], ssem),
                        pltpu.async_copy(b, xg_hbm.at[idx2], ssem))
            nxt = j + 1
            if nbuf <= nxt < nchunk:
                old = nxt - nbuf
                scats[old][0].wait()
                scats[old][1].wait()
                reads[nxt] = pltpu.async_copy(
                    x_hbm.at[pl.ds(base + nxt * _CHUNK, _CHUNK)],
                    rows[nxt % nbuf], rsem)
        for j in range(max(nchunk - nbuf, 0), nchunk):
            scats[j][0].wait()
            scats[j][1].wait()
        cw1.wait()
        cw2.wait()

    return disp(x_flat, pos1, pos2, w1, w2)


def _ffn_body(eb_ref, rb_ref, xg_ref, w1_ref, b1_ref, w2_ref, b2_ref,
              wl_ref, yg_ref):
    h = jnp.dot(xg_ref[...].astype(jnp.bfloat16), w1_ref[0],
                preferred_element_type=jnp.float32) + b1_ref[0]
    h = jnp.maximum(h, 0.0).astype(jnp.bfloat16)
    y = jnp.dot(h, w2_ref[0], preferred_element_type=jnp.float32) + b2_ref[0]
    yg_ref[...] = y * wl_ref[0, 0][:, None]


def _ffn(xg, wl, W1_bf, b1, W2_bf, b2, eb, rb, grid_size):
    grid_spec = pltpu.PrefetchScalarGridSpec(
        num_scalar_prefetch=2,
        grid=(grid_size,),
        in_specs=[
            pl.BlockSpec((_FBM, LATENT), lambda g, eb, rb: (rb[g], 0)),
            pl.BlockSpec((1, LATENT, FFN), lambda g, eb, rb: (eb[g], 0, 0)),
            pl.BlockSpec((1, 1, FFN), lambda g, eb, rb: (eb[g], 0, 0)),
            pl.BlockSpec((1, FFN, LATENT), lambda g, eb, rb: (eb[g], 0, 0)),
            pl.BlockSpec((1, 1, LATENT), lambda g, eb, rb: (eb[g], 0, 0)),
            pl.BlockSpec((1, 1, _FBM), lambda g, eb, rb: (rb[g], 0, 0)),
        ],
        out_specs=pl.BlockSpec((_FBM, LATENT), lambda g, eb, rb: (rb[g], 0)),
    )
    return pl.pallas_call(
        _ffn_body,
        grid_spec=grid_spec,
        out_shape=jax.ShapeDtypeStruct((_RTOT, LATENT), jnp.float32),
        compiler_params=pltpu.CompilerParams(
            dimension_semantics=("arbitrary",),
        ),
    )(eb, rb, xg, W1_bf, b1.reshape(NE, 1, FFN), W2_bf,
      b2.reshape(NE, 1, LATENT), wl.reshape(_RTOT // _FBM, 1, _FBM))


def _combine(yg, pos1, pos2, n):
    per_w = n // _NW
    nchunk = per_w // _CHUNK
    mesh = plsc.VectorSubcoreMesh(core_axis_name="c", subcore_axis_name="s")

    @functools.partial(
        pl.kernel, mesh=mesh,
        out_type=jax.ShapeDtypeStruct((n, LATENT), jnp.float32),
        scratch_types=[
            pltpu.VMEM((per_w,), jnp.int32),
            pltpu.VMEM((per_w,), jnp.int32),
            pltpu.VMEM((_CHUNK, LATENT), jnp.float32),
            pltpu.VMEM((_CHUNK, LATENT), jnp.float32),
            pltpu.VMEM((_CHUNK, LATENT), jnp.float32),
            pltpu.VMEM((_CHUNK, LATENT), jnp.float32),
            pltpu.SemaphoreType.DMA,
            pltpu.SemaphoreType.DMA,
        ],
    )
    def comb(yg_hbm, p1_hbm, p2_hbm, out_hbm, p1_v, p2_v,
             ra0, rb0, ra1, rb1, gsem, wsem):
        ra = (ra0, ra1)
        rb = (rb0, rb1)
        wid = lax.axis_index("s") * _NSC + lax.axis_index("c")
        base = wid * per_w
        pltpu.sync_copy(p1_hbm.at[pl.ds(base, per_w)], p1_v)
        pltpu.sync_copy(p2_hbm.at[pl.ds(base, per_w)], p2_v)

        def gath(j):
            idx1 = p1_v[pl.ds(j * _CHUNK, _CHUNK)]
            idx2 = p2_v[pl.ds(j * _CHUNK, _CHUNK)]
            return (pltpu.async_copy(yg_hbm.at[idx1], ra[j % 2], gsem),
                    pltpu.async_copy(yg_hbm.at[idx2], rb[j % 2], gsem))

        gaths = [None] * nchunk
        writes = [None] * nchunk
        gaths[0] = gath(0)
        for j in range(nchunk):
            gaths[j][0].wait()
            gaths[j][1].wait()
            if j + 1 < nchunk:
                if j - 1 >= 0:
                    writes[j - 1].wait()
                gaths[j + 1] = gath(j + 1)
            a, b = ra[j % 2], rb[j % 2]

            def add_body(c, carry, a=a, b=b):
                for t in range(_CHUNK):
                    a[t, pl.ds(c * 64, 16)] = (a[t, pl.ds(c * 64, 16)]
                                               + b[t, pl.ds(c * 64, 16)])
                    a[t, pl.ds(c * 64 + 16, 16)] = (
                        a[t, pl.ds(c * 64 + 16, 16)]
                        + b[t, pl.ds(c * 64 + 16, 16)])
                    a[t, pl.ds(c * 64 + 32, 16)] = (
                        a[t, pl.ds(c * 64 + 32, 16)]
                        + b[t, pl.ds(c * 64 + 32, 16)])
                    a[t, pl.ds(c * 64 + 48, 16)] = (
                        a[t, pl.ds(c * 64 + 48, 16)]
                        + b[t, pl.ds(c * 64 + 48, 16)])
                return carry

            lax.fori_loop(0, LATENT // 64, add_body, 0)
            writes[j] = pltpu.async_copy(
                a, out_hbm.at[pl.ds(base + j * _CHUNK, _CHUNK)], wsem)
        writes[nchunk - 1].wait()
        if nchunk >= 2:
            writes[nchunk - 2].wait()

    return comb(yg, pos1, pos2)


def kernel(x, gate_W, gate_b, W1, b1, W2, b2):
    B, T, D = x.shape
    n = B * T
    x_flat = x.reshape(n, D)

    pos1, pos2, w1, w2, counts = _gating(x_flat, gate_W, gate_b)
    pos1 = pos1.reshape(n)
    pos2 = pos2.reshape(n)
    w1 = w1.reshape(n)
    w2 = w2.reshape(n)
    counts = counts.reshape(NE)

    xg, wl = _dispatch(x_flat, pos1, pos2, w1, w2)

    grid_size = 2 * n // _FBM + NE
    nblocks = (counts + (_FBM - 1)) // _FBM
    ends = jnp.cumsum(nblocks)
    starts = ends - nblocks
    total = ends[-1]
    g = jnp.arange(grid_size, dtype=jnp.int32)
    eb = jnp.sum((g[:, None] >= starts[None, :]).astype(jnp.int32), axis=1) - 1
    eb = jnp.clip(eb, 0, NE - 1)
    j = g - starts[eb]
    rb = eb * _NBLK + j
    valid = g < total
    eb_last = eb[jnp.maximum(total - 1, 0)]
    eb = jnp.where(valid, eb, eb_last)
    rb = jnp.where(valid, rb, NE * _NBLK)

    W1_bf = W1.astype(jnp.bfloat16)
    W2_bf = W2.astype(jnp.bfloat16)
    yg = _ffn(xg, wl, W1_bf, b1, W2_bf, b2, eb, rb, grid_size)

    out = _combine(yg, pos1, pos2, n)
    return out.reshape(B, T, D)


# speedup vs baseline: 2.0720x; 1.0247x over previous
import functools

import jax
import jax.numpy as jnp
from jax import lax
from jax.experimental import pallas as pl
from jax.experimental.pallas import tpu as pltpu
from jax.experimental.pallas import tpu_sc as plsc

LATENT = 1024
FFN = 4096
NE = 8
EPS = 0.01

CAP = 8192
_GBM = 1024
_FBM = 256
_NBLK = CAP // _FBM
_RTOT = NE * CAP + _FBM

_NSC = 2
_NSUB = 16
_NW = _NSC * _NSUB
_CHUNK = 16


def _gate_body(x_ref, gw_ref, gb_ref, pos1_ref, pos2_ref, w1_ref, w2_ref,
               cnt_ref, xbf_ref, base_ref):
    t = pl.program_id(0)

    @pl.when(t == 0)
    def _init():
        base_ref[...] = jnp.zeros_like(base_ref)

    x_bf = x_ref[...].astype(jnp.bfloat16)
    u = lax.bitcast_convert_type(x_bf, jnp.uint16)
    lo = u[:, :LATENT // 2].astype(jnp.uint32)
    hi = u[:, LATENT // 2:].astype(jnp.uint32)
    xbf_ref[...] = lax.bitcast_convert_type(lo | (hi << 16), jnp.int32)
    logits = jnp.dot(x_bf,
                     gw_ref[...].astype(jnp.bfloat16),
                     preferred_element_type=jnp.float32) + gb_ref[...]
    m = jnp.max(logits, axis=-1, keepdims=True)
    p = jnp.exp(logits - m)
    g = p / jnp.sum(p, axis=-1, keepdims=True)
    iota = lax.broadcasted_iota(jnp.int32, g.shape, 1)
    m1 = jnp.max(g, axis=-1, keepdims=True)
    a1 = jnp.min(jnp.where(g == m1, iota, NE), axis=-1, keepdims=True)
    g2 = jnp.where(iota == a1, -jnp.inf, g)
    m2 = jnp.max(g2, axis=-1, keepdims=True)
    a2 = jnp.min(jnp.where(g2 == m2, iota, NE), axis=-1, keepdims=True)
    denom = m1 + m2 + EPS
    w1_ref[0, 0, :] = (m1 / denom)[:, 0]
    w2_ref[0, 0, :] = (m2 / denom)[:, 0]

    oh1 = (iota == a1).astype(jnp.float32)
    oh2 = (iota == a2).astype(jnp.float32)
    oh = oh1 + oh2
    ir = lax.broadcasted_iota(jnp.int32, (_GBM, _GBM), 0)
    ic = lax.broadcasted_iota(jnp.int32, (_GBM, _GBM), 1)
    tri = (ir > ic).astype(jnp.bfloat16)
    s = jnp.dot(tri, oh.astype(jnp.bfloat16), preferred_element_type=jnp.float32)
    rank1 = jnp.sum(s * oh1, axis=1).astype(jnp.int32)
    rank2 = jnp.sum(s * oh2, axis=1).astype(jnp.int32)

    base = base_ref[...].astype(jnp.float32)
    base1 = jnp.sum(base * oh1, axis=1).astype(jnp.int32)
    base2 = jnp.sum(base * oh2, axis=1).astype(jnp.int32)
    pos1_ref[0, 0, :] = a1[:, 0] * CAP + base1 + rank1
    pos2_ref[0, 0, :] = a2[:, 0] * CAP + base2 + rank2

    new_base = base_ref[...] + jnp.sum(oh, axis=0, keepdims=True).astype(jnp.int32)
    base_ref[...] = new_base
    cnt_ref[...] = new_base


def _gating(x_flat, gate_W, gate_b):
    n = x_flat.shape[0]
    nb = n // _GBM
    return pl.pallas_call(
        _gate_body,
        grid=(nb,),
        in_specs=[
            pl.BlockSpec((_GBM, LATENT), lambda t: (t, 0)),
            pl.BlockSpec((LATENT, NE), lambda t: (0, 0)),
            pl.BlockSpec((1, NE), lambda t: (0, 0)),
        ],
        out_specs=[
            pl.BlockSpec((1, 1, _GBM), lambda t: (t, 0, 0)),
            pl.BlockSpec((1, 1, _GBM), lambda t: (t, 0, 0)),
            pl.BlockSpec((1, 1, _GBM), lambda t: (t, 0, 0)),
            pl.BlockSpec((1, 1, _GBM), lambda t: (t, 0, 0)),
            pl.BlockSpec((1, NE), lambda t: (0, 0)),
            pl.BlockSpec((_GBM, LATENT // 2), lambda t: (t, 0)),
        ],
        out_shape=[
            jax.ShapeDtypeStruct((nb, 1, _GBM), jnp.int32),
            jax.ShapeDtypeStruct((nb, 1, _GBM), jnp.int32),
            jax.ShapeDtypeStruct((nb, 1, _GBM), jnp.float32),
            jax.ShapeDtypeStruct((nb, 1, _GBM), jnp.float32),
            jax.ShapeDtypeStruct((1, NE), jnp.int32),
            jax.ShapeDtypeStruct((n, LATENT // 2), jnp.int32),
        ],
        scratch_shapes=[pltpu.VMEM((1, NE), jnp.int32)],
        compiler_params=pltpu.CompilerParams(
            dimension_semantics=("arbitrary",),
        ),
    )(x_flat, gate_W, gate_b.reshape(1, NE))


def _dispatch(x_flat, pos1, pos2, w1, w2):
    n = x_flat.shape[0]
    per_w = n // _NW
    nchunk = per_w // _CHUNK
    mesh = plsc.VectorSubcoreMesh(core_axis_name="c", subcore_axis_name="s")

    nbuf = 4

    @functools.partial(
        pl.kernel, mesh=mesh,
        out_type=[
            jax.ShapeDtypeStruct((_RTOT, LATENT // 2), jnp.int32),
            jax.ShapeDtypeStruct((_RTOT,), jnp.float32),
        ],
        scratch_types=[
            pltpu.VMEM((per_w,), jnp.int32),
            pltpu.VMEM((per_w,), jnp.int32),
            pltpu.VMEM((per_w,), jnp.float32),
            pltpu.VMEM((per_w,), jnp.float32),
        ] + [pltpu.VMEM((_CHUNK, LATENT // 2), jnp.int32) for _ in range(nbuf)]
          + [pltpu.SemaphoreType.DMA, pltpu.SemaphoreType.DMA,
             pltpu.SemaphoreType.DMA],
    )
    def disp(x_hbm, p1_hbm, p2_hbm, w1_hbm, w2_hbm, xg_hbm, wl_hbm,
             p1_v, p2_v, w1_v, w2_v, *bufs_and_sems):
        rows = bufs_and_sems[:nbuf]
        rsem, ssem, wsem = bufs_and_sems[nbuf:]
        wid = lax.axis_index("s") * _NSC + lax.axis_index("c")
        base = wid * per_w
        pltpu.sync_copy(p1_hbm.at[pl.ds(base, per_w)], p1_v)
        pltpu.sync_copy(p2_hbm.at[pl.ds(base, per_w)], p2_v)
        pltpu.sync_copy(w1_hbm.at[pl.ds(base, per_w)], w1_v)
        pltpu.sync_copy(w2_hbm.at[pl.ds(base, per_w)], w2_v)
        cw1 = pltpu.async_copy(w1_v, wl_hbm.at[p1_v], wsem)
        cw2 = pltpu.async_copy(w2_v, wl_hbm.at[p2_v], wsem)

        reads = [None] * nchunk
        scats = [None] * nchunk
        for j in range(min(nbuf, nchunk)):
            reads[j] = pltpu.async_copy(
                x_hbm.at[pl.ds(base + j * _CHUNK, _CHUNK)], rows[j % nbuf],
                rsem)
        for j in range(nchunk):
            reads[j].wait()
            idx1 = p1_v[pl.ds(j * _CHUNK, _CHUNK)]
            idx2 = p2_v[pl.ds(j * _CHUNK, _CHUNK)]
            b = rows[j % nbuf]
            scats[j] = (pltpu.async_copy(b, xg_hbm.at[idx1], ssem),
                        pltpu.async_copy(b, xg_hbm.at[idx2], ssem))
            nxt = j + 1
            if nbuf <= nxt < nchunk:
                old = nxt - nbuf
                scats[old][0].wait()
                scats[old][1].wait()
                reads[nxt] = pltpu.async_copy(
                    x_hbm.at[pl.ds(base + nxt * _CHUNK, _CHUNK)],
                    rows[nxt % nbuf], rsem)
        for j in range(max(nchunk - nbuf, 0), nchunk):
            scats[j][0].wait()
            scats[j][1].wait()
        cw1.wait()
        cw2.wait()

    return disp(x_flat, pos1, pos2, w1, w2)


def _ffn_body(eb_ref, rb_ref, xg_ref, w1_ref, b1_ref, w2_ref, b2_ref,
              wl_ref, yg_ref):
    w = lax.bitcast_convert_type(xg_ref[...], jnp.uint32)
    lo = (w & 0xFFFF).astype(jnp.uint16)
    hi = (w >> 16).astype(jnp.uint16)
    x_bf = lax.bitcast_convert_type(
        jnp.concatenate([lo, hi], axis=1), jnp.bfloat16)
    h = jnp.dot(x_bf, w1_ref[0],
                preferred_element_type=jnp.float32) + b1_ref[0]
    h = jnp.maximum(h, 0.0).astype(jnp.bfloat16)
    y = jnp.dot(h, w2_ref[0], preferred_element_type=jnp.float32) + b2_ref[0]
    yg_ref[...] = y * wl_ref[0, 0][:, None]


def _ffn(xg, wl, W1_bf, b1, W2_bf, b2, eb, rb, grid_size):
    grid_spec = pltpu.PrefetchScalarGridSpec(
        num_scalar_prefetch=2,
        grid=(grid_size,),
        in_specs=[
            pl.BlockSpec((_FBM, LATENT // 2), lambda g, eb, rb: (rb[g], 0)),
            pl.BlockSpec((1, LATENT, FFN), lambda g, eb, rb: (eb[g], 0, 0)),
            pl.BlockSpec((1, 1, FFN), lambda g, eb, rb: (eb[g], 0, 0)),
            pl.BlockSpec((1, FFN, LATENT), lambda g, eb, rb: (eb[g], 0, 0)),
            pl.BlockSpec((1, 1, LATENT), lambda g, eb, rb: (eb[g], 0, 0)),
            pl.BlockSpec((1, 1, _FBM), lambda g, eb, rb: (rb[g], 0, 0)),
        ],
        out_specs=pl.BlockSpec((_FBM, LATENT), lambda g, eb, rb: (rb[g], 0)),
    )
    return pl.pallas_call(
        _ffn_body,
        grid_spec=grid_spec,
        out_shape=jax.ShapeDtypeStruct((_RTOT, LATENT), jnp.float32),
        compiler_params=pltpu.CompilerParams(
            dimension_semantics=("arbitrary",),
        ),
    )(eb, rb, xg, W1_bf, b1.reshape(NE, 1, FFN), W2_bf,
      b2.reshape(NE, 1, LATENT), wl.reshape(_RTOT // _FBM, 1, _FBM))


def _combine(yg, pos1, pos2, n):
    per_w = n // _NW
    nchunk = per_w // _CHUNK
    mesh = plsc.VectorSubcoreMesh(core_axis_name="c", subcore_axis_name="s")

    @functools.partial(
        pl.kernel, mesh=mesh,
        out_type=jax.ShapeDtypeStruct((n, LATENT), jnp.float32),
        scratch_types=[
            pltpu.VMEM((per_w,), jnp.int32),
            pltpu.VMEM((per_w,), jnp.int32),
            pltpu.VMEM((_CHUNK, LATENT), jnp.float32),
            pltpu.VMEM((_CHUNK, LATENT), jnp.float32),
            pltpu.VMEM((_CHUNK, LATENT), jnp.float32),
            pltpu.VMEM((_CHUNK, LATENT), jnp.float32),
            pltpu.SemaphoreType.DMA,
            pltpu.SemaphoreType.DMA,
        ],
    )
    def comb(yg_hbm, p1_hbm, p2_hbm, out_hbm, p1_v, p2_v,
             ra0, rb0, ra1, rb1, gsem, wsem):
        ra = (ra0, ra1)
        rb = (rb0, rb1)
        wid = lax.axis_index("s") * _NSC + lax.axis_index("c")
        base = wid * per_w
        pltpu.sync_copy(p1_hbm.at[pl.ds(base, per_w)], p1_v)
        pltpu.sync_copy(p2_hbm.at[pl.ds(base, per_w)], p2_v)

        def gath(j):
            idx1 = p1_v[pl.ds(j * _CHUNK, _CHUNK)]
            idx2 = p2_v[pl.ds(j * _CHUNK, _CHUNK)]
            return (pltpu.async_copy(yg_hbm.at[idx1], ra[j % 2], gsem),
                    pltpu.async_copy(yg_hbm.at[idx2], rb[j % 2], gsem))

        gaths = [None] * nchunk
        writes = [None] * nchunk
        gaths[0] = gath(0)
        for j in range(nchunk):
            gaths[j][0].wait()
            gaths[j][1].wait()
            if j + 1 < nchunk:
                if j - 1 >= 0:
                    writes[j - 1].wait()
                gaths[j + 1] = gath(j + 1)
            a, b = ra[j % 2], rb[j % 2]

            def add_body(c, carry, a=a, b=b):
                for t in range(_CHUNK):
                    a[t, pl.ds(c * 64, 16)] = (a[t, pl.ds(c * 64, 16)]
                                               + b[t, pl.ds(c * 64, 16)])
                    a[t, pl.ds(c * 64 + 16, 16)] = (
                        a[t, pl.ds(c * 64 + 16, 16)]
                        + b[t, pl.ds(c * 64 + 16, 16)])
                    a[t, pl.ds(c * 64 + 32, 16)] = (
                        a[t, pl.ds(c * 64 + 32, 16)]
                        + b[t, pl.ds(c * 64 + 32, 16)])
                    a[t, pl.ds(c * 64 + 48, 16)] = (
                        a[t, pl.ds(c * 64 + 48, 16)]
                        + b[t, pl.ds(c * 64 + 48, 16)])
                return carry

            lax.fori_loop(0, LATENT // 64, add_body, 0)
            writes[j] = pltpu.async_copy(
                a, out_hbm.at[pl.ds(base + j * _CHUNK, _CHUNK)], wsem)
        writes[nchunk - 1].wait()
        if nchunk >= 2:
            writes[nchunk - 2].wait()

    return comb(yg, pos1, pos2)


def kernel(x, gate_W, gate_b, W1, b1, W2, b2):
    B, T, D = x.shape
    n = B * T
    x_flat = x.reshape(n, D)

    pos1, pos2, w1, w2, counts, x_bf = _gating(x_flat, gate_W, gate_b)
    pos1 = pos1.reshape(n)
    pos2 = pos2.reshape(n)
    w1 = w1.reshape(n)
    w2 = w2.reshape(n)
    counts = counts.reshape(NE)

    xg, wl = _dispatch(x_bf, pos1, pos2, w1, w2)

    grid_size = 2 * n // _FBM + NE
    nblocks = (counts + (_FBM - 1)) // _FBM
    ends = jnp.cumsum(nblocks)
    starts = ends - nblocks
    total = ends[-1]
    g = jnp.arange(grid_size, dtype=jnp.int32)
    eb = jnp.sum((g[:, None] >= starts[None, :]).astype(jnp.int32), axis=1) - 1
    eb = jnp.clip(eb, 0, NE - 1)
    j = g - starts[eb]
    rb = eb * _NBLK + j
    valid = g < total
    eb_last = eb[jnp.maximum(total - 1, 0)]
    eb = jnp.where(valid, eb, eb_last)
    rb = jnp.where(valid, rb, NE * _NBLK)

    W1_bf = W1.astype(jnp.bfloat16)
    W2_bf = W2.astype(jnp.bfloat16)
    yg = _ffn(xg, wl, W1_bf, b1, W2_bf, b2, eb, rb, grid_size)

    out = _combine(yg, pos1, pos2, n)
    return out.reshape(B, T, D)
